# Initial kernel scaffold; baseline (speedup 1.0000x reference)
import jax
import jax.numpy as jnp
from jax.experimental import pallas as pl


def kernel(length, node_features, node_attributes, edge_attributes, edge_index, W1, W2, W3, W4, L0, L1, Wsc):
    raise NotImplementedError("write your pallas kernel here")


# speedup vs baseline: 1.6207x; 1.6207x over previous
import functools

import jax
import jax.numpy as jnp
import numpy as np
from jax import lax
from jax.experimental import pallas as pl
from jax.experimental.pallas import tpu as pltpu
from jax.experimental.pallas import tpu_sc as plsc

N_NODES = 10000
N_EDGES = 320000
MUL = 128
ACT_NORM = 1.6790

_EBLK = 2000


def _mlp_body(len_ref, w1_ref, w2_ref, w3_ref, w4_ref, out_ref):
    x = len_ref[...]
    h = jax.nn.silu(jnp.dot(x, w1_ref[...], preferred_element_type=jnp.float32)) * ACT_NORM
    h = jax.nn.silu(jnp.dot(h, w2_ref[...], preferred_element_type=jnp.float32)) * ACT_NORM
    h = jax.nn.silu(jnp.dot(h, w3_ref[...], preferred_element_type=jnp.float32)) * ACT_NORM
    t = jnp.dot(h, w4_ref[...], preferred_element_type=jnp.float32)
    out_ref[0] = t[:, :MUL]
    out_ref[1] = t[:, MUL:]


def _radial_mlp(length, W1s, W2s, W3s, W4s):
    n_blk = N_EDGES // _EBLK
    return pl.pallas_call(
        _mlp_body,
        grid=(n_blk,),
        in_specs=[
            pl.BlockSpec((_EBLK, 8), lambda i: (i, 0)),
            pl.BlockSpec(W1s.shape, lambda i: (0, 0)),
            pl.BlockSpec(W2s.shape, lambda i: (0, 0)),
            pl.BlockSpec(W3s.shape, lambda i: (0, 0)),
            pl.BlockSpec(W4s.shape, lambda i: (0, 0)),
        ],
        out_specs=pl.BlockSpec((2, _EBLK, MUL), lambda i: (0, i, 0)),
        out_shape=jax.ShapeDtypeStruct((2, N_EDGES, MUL), jnp.float32),
    )(length, W1s, W2s, W3s, W4s)


_NS = 16
_EPT = N_EDGES // _NS
_NC = 80
_NFULL = _EPT // _NC
_NPAD = 10240
_RPT = _NPAD // _NS
_ZR = 80


def _sc_scatter(sender, receiver, node_features, w_packed, sh_t):
    mesh = plsc.VectorSubcoreMesh(core_axis_name="c", subcore_axis_name="s")
    k = functools.partial(
        pl.kernel,
        out_type=jax.ShapeDtypeStruct((4, _NPAD, MUL), jnp.float32),
        mesh=mesh,
        scratch_types=[
            pltpu.VMEM((2, _NC), jnp.int32),
            pltpu.VMEM((1, _NC), jnp.float32),
            pltpu.VMEM((_NC, MUL), jnp.float32),
            pltpu.VMEM((_NC, MUL), jnp.float32),
            pltpu.VMEM_SHARED((_NPAD, MUL), jnp.float32),
            pltpu.SemaphoreType.DMA,
        ],
    )(_sc_body_fixed)
    return k(sender, receiver, node_features, w_packed, sh_t)


def _sc_body_fixed(snd_hbm, rcv_hbm, nf_hbm, w_hbm, sh_hbm, out_hbm,
                   idx_v, s_v, w_v, xs_v, acc, sem):
    c = lax.axis_index("c")
    s = lax.axis_index("s")
    tile_base = s * _EPT
    zvec = jnp.zeros((16,), jnp.float32)

    def _zero_acc():
        def _zb(r, carry):
            for j in range(MUL // 16):
                xs_v[r, pl.ds(j * 16, 16)] = zvec
            return carry
        lax.fori_loop(0, _ZR, _zb, None)

        def _za(k2, carry):
            pltpu.sync_copy(xs_v, acc.at[pl.ds(s * _RPT + k2 * _ZR, _ZR), :])
            return carry
        lax.fori_loop(0, _RPT // _ZR, _za, None)

    def _chunk(base, blk, wsel, nc, iv, sv, wv, xv):
        pltpu.sync_copy(snd_hbm.at[pl.ds(base, nc)], iv.at[0])
        pltpu.sync_copy(rcv_hbm.at[pl.ds(base, nc)], iv.at[1])
        pltpu.sync_copy(sh_hbm.at[pl.ds(blk * N_EDGES + base, nc)], sv.at[0])
        pltpu.sync_copy(w_hbm.at[wsel, pl.ds(base, nc), :], wv)
        pltpu.async_copy(nf_hbm.at[iv.at[0]], xv, sem).wait()

        def _group(g, carry):
            cvec = sv[0, pl.ds(g * 16, 16)]
            base_e = g * 16
            for l in range(16):
                e = base_e + l
                coef = cvec[l]
                for j in range(MUL // 16):
                    sl = pl.ds(j * 16, 16)
                    xv[e, sl] = wv[e, sl] * xv[e, sl] * coef
            return carry
        lax.fori_loop(0, nc // 16, _group, None)

        pltpu.sync_copy(xv, acc.at[iv.at[1]], add=True)

    _zero_acc()
    plsc.subcore_barrier()

    for p in range(2):
        blk = c * 2 + p
        wsel = jnp.minimum(blk, 1)

        def _do_full(i, carry):
            _chunk(tile_base + i * _NC, blk, wsel, _NC, idx_v, s_v, w_v, xs_v)
            return carry
        lax.fori_loop(0, _NFULL, _do_full, None)

        plsc.subcore_barrier()
        pltpu.sync_copy(acc.at[pl.ds(s * _RPT, _RPT), :],
                        out_hbm.at[blk, pl.ds(s * _RPT, _RPT), :])
        if p == 0:
            _zero_acc()
            plsc.subcore_barrier()


_NBLK = 400


def _tail_body(msg_ref, na_ref, nf_ref, l0_ref, l1_ref, wsc_ref, mout_ref, sc_ref):
    mout_ref[0] = jnp.dot(msg_ref[0], l0_ref[...], preferred_element_type=jnp.float32)
    for m in range(1, 4):
        mout_ref[m] = jnp.dot(msg_ref[m], l1_ref[...], preferred_element_type=jnp.float32)
    na = na_ref[...]
    nv = nf_ref[...]
    t = jnp.concatenate([na[:, a:a + 1] * nv for a in range(4)], axis=1)
    sc_ref[...] = jnp.dot(t, wsc_ref[...], preferred_element_type=jnp.float32)


def _tail(msg, node_attributes, node_features, L0s, L1s, Wsc2):
    n_blk = N_NODES // _NBLK
    return pl.pallas_call(
        _tail_body,
        grid=(n_blk,),
        in_specs=[
            pl.BlockSpec((4, _NBLK, MUL), lambda i: (0, i, 0)),
            pl.BlockSpec((_NBLK, 4), lambda i: (i, 0)),
            pl.BlockSpec((_NBLK, MUL), lambda i: (i, 0)),
            pl.BlockSpec(L0s.shape, lambda i: (0, 0)),
            pl.BlockSpec(L1s.shape, lambda i: (0, 0)),
            pl.BlockSpec(Wsc2.shape, lambda i: (0, 0)),
        ],
        out_specs=[
            pl.BlockSpec((4, _NBLK, MUL), lambda i: (0, i, 0)),
            pl.BlockSpec((_NBLK, MUL), lambda i: (i, 0)),
        ],
        out_shape=[
            jax.ShapeDtypeStruct((4, N_NODES, MUL), jnp.float32),
            jax.ShapeDtypeStruct((N_NODES, MUL), jnp.float32),
        ],
    )(msg, node_attributes, node_features, L0s, L1s, Wsc2)


def kernel(length, node_features, node_attributes, edge_attributes, edge_index,
           W1, W2, W3, W4, L0, L1, Wsc):
    W1s = W1 / np.sqrt(W1.shape[0])
    W2s = W2 / np.sqrt(W2.shape[0])
    W3s = W3 / np.sqrt(W3.shape[0])
    W4s = W4 / np.sqrt(W4.shape[0])
    L0s = L0 / np.sqrt(MUL)
    L1s = L1 / np.sqrt(MUL)
    Wsc2 = Wsc.reshape(4 * MUL, MUL) / np.sqrt(4 * MUL)
    sh_t = edge_attributes.T

    w_packed = _radial_mlp(length, W1s, W2s, W3s, W4s)
    sender = edge_index[0]
    receiver = edge_index[1]
    msg = _sc_scatter(sender, receiver, node_features, w_packed,
                      sh_t.reshape(-1))[:, :N_NODES, :]
    mout, sc0 = _tail(msg, node_attributes, node_features, L0s, L1s, Wsc2)

    msg_reshaped = jnp.transpose(mout, (1, 2, 0))
    sc = jnp.concatenate(
        [sc0, jnp.zeros((N_NODES, 3 * MUL), jnp.float32)], axis=1)
    return (msg_reshaped, sc)


# speedup vs baseline: 2.3293x; 1.4372x over previous
import functools

import jax
import jax.numpy as jnp
import numpy as np
from jax import lax
from jax.experimental import pallas as pl
from jax.experimental.pallas import tpu as pltpu
from jax.experimental.pallas import tpu_sc as plsc

N_NODES = 10000
N_EDGES = 320000
MUL = 128
ACT_NORM = 1.6790

_EBLK = 2000


def _mlp_body(len_ref, w1_ref, w2_ref, w3_ref, w4_ref, out_ref):
    x = len_ref[...]
    h = jax.nn.silu(jnp.dot(x, w1_ref[...], preferred_element_type=jnp.float32)) * ACT_NORM
    h = jax.nn.silu(jnp.dot(h, w2_ref[...], preferred_element_type=jnp.float32)) * ACT_NORM
    h = jax.nn.silu(jnp.dot(h, w3_ref[...], preferred_element_type=jnp.float32)) * ACT_NORM
    t = jnp.dot(h, w4_ref[...], preferred_element_type=jnp.float32)
    out_ref[0] = t[:, :MUL]
    out_ref[1] = t[:, MUL:]


def _radial_mlp(length, W1s, W2s, W3s, W4s):
    n_blk = N_EDGES // _EBLK
    return pl.pallas_call(
        _mlp_body,
        grid=(n_blk,),
        in_specs=[
            pl.BlockSpec((_EBLK, 8), lambda i: (i, 0)),
            pl.BlockSpec(W1s.shape, lambda i: (0, 0)),
            pl.BlockSpec(W2s.shape, lambda i: (0, 0)),
            pl.BlockSpec(W3s.shape, lambda i: (0, 0)),
            pl.BlockSpec(W4s.shape, lambda i: (0, 0)),
        ],
        out_specs=pl.BlockSpec((2, _EBLK, MUL), lambda i: (0, i, 0)),
        out_shape=jax.ShapeDtypeStruct((2, N_EDGES, MUL), jnp.float32),
    )(length, W1s, W2s, W3s, W4s)


_NS = 16
_EPT = N_EDGES // _NS
_NC = 80
_NCHUNK = _EPT // _NC
_NPAD = 10240
_RPT = _NPAD // _NS
_ZR = 80
_NPAIR = _NCHUNK // 2


def _sc_scatter(sender, receiver, node_features, w_packed, sh_flat):
    mesh = plsc.VectorSubcoreMesh(core_axis_name="c", subcore_axis_name="s")
    sets = []
    for _ in range(2):
        sets += [
            pltpu.VMEM((2, _NC), jnp.int32),
            pltpu.VMEM((1, _NC), jnp.float32),
            pltpu.VMEM((_NC, MUL), jnp.float32),
            pltpu.VMEM((_NC, MUL), jnp.float32),
        ]
    k = functools.partial(
        pl.kernel,
        out_type=jax.ShapeDtypeStruct((4, _NPAD, MUL), jnp.float32),
        mesh=mesh,
        scratch_types=sets + [
            pltpu.VMEM_SHARED((_NPAD, MUL), jnp.float32),
        ] + [pltpu.SemaphoreType.DMA] * 10,
    )(_sc_body)
    return k(sender, receiver, node_features, w_packed, sh_flat)


def _sc_body(snd_hbm, rcv_hbm, nf_hbm, w_hbm, sh_hbm, out_hbm,
             iv0, sv0, wv0, xv0, iv1, sv1, wv1, xv1, acc, *sems):
    core = lax.axis_index("c")
    s = lax.axis_index("s")
    tile_base = s * _EPT
    zvec = jnp.zeros((16,), jnp.float32)

    S = [
        dict(iv=iv0, sv=sv0, wv=wv0, xv=xv0,
             ei=sems[0], sh=sems[1], w=sems[2], g=sems[3], sc=sems[4]),
        dict(iv=iv1, sv=sv1, wv=wv1, xv=xv1,
             ei=sems[5], sh=sems[6], w=sems[7], g=sems[8], sc=sems[9]),
    ]

    def _zero_acc():
        def _zb(r, carry):
            for j in range(MUL // 16):
                xv0[r, pl.ds(j * 16, 16)] = zvec
            return carry
        lax.fori_loop(0, _ZR, _zb, None)

        def _za(k2, carry):
            pltpu.sync_copy(xv0, acc.at[pl.ds(s * _RPT + k2 * _ZR, _ZR), :])
            return carry
        lax.fori_loop(0, _RPT // _ZR, _za, None)

    def _lin_issue(base, blk, wsel, t):
        return [
            pltpu.async_copy(snd_hbm.at[pl.ds(base, _NC)], t["iv"].at[0], t["ei"]),
            pltpu.async_copy(rcv_hbm.at[pl.ds(base, _NC)], t["iv"].at[1], t["ei"]),
            pltpu.async_copy(sh_hbm.at[pl.ds(blk * N_EDGES + base, _NC)],
                             t["sv"].at[0], t["sh"]),
            pltpu.async_copy(w_hbm.at[wsel, pl.ds(base, _NC), :], t["wv"], t["w"]),
        ]

    def _compute(t):
        sv, wv, xv = t["sv"], t["wv"], t["xv"]

        def _group(g, carry):
            cvec = sv[0, pl.ds(g * 16, 16)]
            base_e = g * 16
            for l in range(16):
                e = base_e + l
                coef = cvec[l]
                for j in range(MUL // 16):
                    sl = pl.ds(j * 16, 16)
                    xv[e, sl] = wv[e, sl] * xv[e, sl] * coef
            return carry
        lax.fori_loop(0, _NC // 16, _group, None)

    _zero_acc()
    plsc.subcore_barrier()

    def _pass(p, carry):
        blk = core * 2 + p
        wsel = jnp.minimum(blk, 1)

        def _pair(i, carry2):
            b0 = tile_base + 2 * i * _NC
            b1 = b0 + _NC
            dA = _lin_issue(b0, blk, wsel, S[0])
            dB = _lin_issue(b1, blk, wsel, S[1])
            for d in dA:
                d.wait()
            gA = pltpu.async_copy(nf_hbm.at[S[0]["iv"].at[0]], S[0]["xv"], S[0]["g"])
            for d in dB:
                d.wait()
            gB = pltpu.async_copy(nf_hbm.at[S[1]["iv"].at[0]], S[1]["xv"], S[1]["g"])
            gA.wait()
            _compute(S[0])
            sA = pltpu.async_copy(S[0]["xv"], acc.at[S[0]["iv"].at[1]],
                                  S[0]["sc"], add=True)
            gB.wait()
            _compute(S[1])
            sB = pltpu.async_copy(S[1]["xv"], acc.at[S[1]["iv"].at[1]],
                                  S[1]["sc"], add=True)
            sA.wait()
            sB.wait()
            return carry2
        lax.fori_loop(0, _NPAIR, _pair, None)

        plsc.subcore_barrier()
        pltpu.sync_copy(acc.at[pl.ds(s * _RPT, _RPT), :],
                        out_hbm.at[blk, pl.ds(s * _RPT, _RPT), :])

        @pl.when(p == 0)
        def _():
            _zero_acc()
        plsc.subcore_barrier()
        return carry

    lax.fori_loop(0, 2, _pass, None)


_NBLK = 400


def _tail_body(msg_ref, na_ref, nf_ref, l0_ref, l1_ref, wsc_ref, mout_ref, sc_ref):
    mout_ref[0] = jnp.dot(msg_ref[0], l0_ref[...], preferred_element_type=jnp.float32)
    for m in range(1, 4):
        mout_ref[m] = jnp.dot(msg_ref[m], l1_ref[...], preferred_element_type=jnp.float32)
    na = na_ref[...]
    nv = nf_ref[...]
    t = jnp.concatenate([na[:, a:a + 1] * nv for a in range(4)], axis=1)
    sc_ref[...] = jnp.dot(t, wsc_ref[...], preferred_element_type=jnp.float32)


def _tail(msg, node_attributes, node_features, L0s, L1s, Wsc2):
    n_blk = N_NODES // _NBLK
    return pl.pallas_call(
        _tail_body,
        grid=(n_blk,),
        in_specs=[
            pl.BlockSpec((4, _NBLK, MUL), lambda i: (0, i, 0)),
            pl.BlockSpec((_NBLK, 4), lambda i: (i, 0)),
            pl.BlockSpec((_NBLK, MUL), lambda i: (i, 0)),
            pl.BlockSpec(L0s.shape, lambda i: (0, 0)),
            pl.BlockSpec(L1s.shape, lambda i: (0, 0)),
            pl.BlockSpec(Wsc2.shape, lambda i: (0, 0)),
        ],
        out_specs=[
            pl.BlockSpec((4, _NBLK, MUL), lambda i: (0, i, 0)),
            pl.BlockSpec((_NBLK, MUL), lambda i: (i, 0)),
        ],
        out_shape=[
            jax.ShapeDtypeStruct((4, N_NODES, MUL), jnp.float32),
            jax.ShapeDtypeStruct((N_NODES, MUL), jnp.float32),
        ],
    )(msg, node_attributes, node_features, L0s, L1s, Wsc2)


def kernel(length, node_features, node_attributes, edge_attributes, edge_index,
           W1, W2, W3, W4, L0, L1, Wsc):
    W1s = W1 / np.sqrt(W1.shape[0])
    W2s = W2 / np.sqrt(W2.shape[0])
    W3s = W3 / np.sqrt(W3.shape[0])
    W4s = W4 / np.sqrt(W4.shape[0])
    L0s = L0 / np.sqrt(MUL)
    L1s = L1 / np.sqrt(MUL)
    Wsc2 = Wsc.reshape(4 * MUL, MUL) / np.sqrt(4 * MUL)
    sh_t = edge_attributes.T

    w_packed = _radial_mlp(length, W1s, W2s, W3s, W4s)
    msg = _sc_scatter(edge_index[0], edge_index[1], node_features, w_packed,
                      sh_t.reshape(-1))[:, :N_NODES, :]
    mout, sc0 = _tail(msg, node_attributes, node_features, L0s, L1s, Wsc2)

    msg_reshaped = jnp.transpose(mout, (1, 2, 0))
    sc = jnp.concatenate(
        [sc0, jnp.zeros((N_NODES, 3 * MUL), jnp.float32)], axis=1)
    return (msg_reshaped, sc)
